# baseline (speedup 1.0000x reference)
import functools

import jax
import jax.numpy as jnp
from jax import lax
from jax.experimental import pallas as pl
from jax.experimental.pallas import tpu as pltpu
from jax.experimental.pallas import tpu_sc as plsc

N = 10000
E = 160000
IN_DIM = 256
HIDDEN = 512
NUM_CLASSES = 128

NC = 2
NS = 16
DB = 128
EB = 64
N_ACC = 10240
N_PAD_ROWS = N_ACC - N

EPT16 = 10240
EPT32 = 5120
NB = 40

STRIP = N_ACC // NS
STAGE_ROWS = 40

_MESH = dict(core_axis_name="c", subcore_axis_name="s")


def _row_span(s):
    start = s * STRIP
    nch = jnp.where(s < NS - 1, STRIP // STAGE_ROWS,
                    (N - (NS - 1) * STRIP) // STAGE_ROWS)
    return start, nch


def _sc_degree():
    nb = EPT32 // DB

    @functools.partial(
        pl.kernel,
        out_type=jax.ShapeDtypeStruct((NC * N_ACC,), jnp.float32),
        mesh=plsc.VectorSubcoreMesh(**_MESH),
        scratch_types=[
            pltpu.VMEM((nb, DB), jnp.int32),
            pltpu.VMEM((DB,), jnp.float32),
            pltpu.VMEM((STRIP,), jnp.float32),
            pltpu.VMEM_SHARED((N_ACC,), jnp.float32),
        ],
    )
    def deg_kernel(dst_hbm, out_hbm, idx_v, ones_v, stage_v, acc_sh):
        c = lax.axis_index("c")
        s = lax.axis_index("s")
        wid = c * NS + s
        for i in range(DB // 16):
            ones_v[pl.ds(i * 16, 16)] = jnp.ones((16,), jnp.float32)
        for i in range(STRIP // 16):
            stage_v[pl.ds(i * 16, 16)] = jnp.zeros((16,), jnp.float32)
        pltpu.sync_copy(dst_hbm.at[wid], idx_v)
        pltpu.sync_copy(stage_v, acc_sh.at[pl.ds(s * STRIP, STRIP)])
        plsc.subcore_barrier()

        def body(b, carry):
            pltpu.sync_copy(ones_v, acc_sh.at[idx_v.at[b]], add=True)
            return carry

        lax.fori_loop(0, nb, body, 0)
        plsc.subcore_barrier()
        pltpu.sync_copy(acc_sh.at[pl.ds(s * STRIP, STRIP)], stage_v)
        pltpu.sync_copy(stage_v, out_hbm.at[pl.ds(c * N_ACC + s * STRIP, STRIP)])

    return deg_kernel


def _edge_ring(y_view, src_v, dst_v, bufs, gsems, ssems, acc_sh, nb):
    pltpu.async_copy(y_view.at[src_v.at[0]], bufs[0], gsems[0])
    pltpu.async_copy(y_view.at[src_v.at[1]], bufs[1], gsems[1])

    def body(j, carry):
        for k in range(4):
            b = 4 * j + k
            k2 = (k + 2) % 4
            pltpu.make_async_copy(y_view.at[src_v.at[b]], bufs[k], gsems[k]).wait()
            pltpu.async_copy(bufs[k], acc_sh.at[dst_v.at[b]], ssems[k], add=True)

            @pl.when(b >= 2)
            def _drain():
                pltpu.make_async_copy(
                    bufs[k2], acc_sh.at[dst_v.at[b]], ssems[k2]).wait()

            @pl.when(b + 2 < nb)
            def _prefetch():
                pltpu.async_copy(y_view.at[src_v.at[b + 2]], bufs[k2], gsems[k2])

        return carry

    lax.fori_loop(0, nb // 4, body, 0)
    pltpu.make_async_copy(
        bufs[(nb - 2) % 4], acc_sh.at[dst_v.at[0]], ssems[(nb - 2) % 4]).wait()
    pltpu.make_async_copy(
        bufs[(nb - 1) % 4], acc_sh.at[dst_v.at[0]], ssems[(nb - 1) % 4]).wait()


def _stage_rows(src_at_rows, dst_at_rows, stage_v, s):
    start, nch = _row_span(s)

    def body(r, carry):
        row0 = start + r * STAGE_ROWS
        pltpu.sync_copy(src_at_rows(row0), stage_v)
        pltpu.sync_copy(stage_v, dst_at_rows(row0))
        return carry

    lax.fori_loop(0, nch, body, 0)


_PROP_SCRATCH = [
    pltpu.VMEM((NB, EB), jnp.int32),
    pltpu.VMEM((NB, EB), jnp.int32),
    pltpu.VMEM((EB, 128), jnp.float32),
    pltpu.VMEM((EB, 128), jnp.float32),
    pltpu.VMEM((EB, 128), jnp.float32),
    pltpu.VMEM((EB, 128), jnp.float32),
    pltpu.VMEM_SHARED((N_ACC, 128), jnp.float32),
    pltpu.SemaphoreType.DMA,
    pltpu.SemaphoreType.DMA,
    pltpu.SemaphoreType.DMA,
    pltpu.SemaphoreType.DMA,
    pltpu.SemaphoreType.DMA,
    pltpu.SemaphoreType.DMA,
    pltpu.SemaphoreType.DMA,
    pltpu.SemaphoreType.DMA,
]


def _sc_propagate_l1():
    kc, kcpc = 4, 2

    @functools.partial(
        pl.kernel,
        out_type=jax.ShapeDtypeStruct((kc, N, 128), jnp.float32),
        mesh=plsc.VectorSubcoreMesh(**_MESH),
        scratch_types=list(_PROP_SCRATCH),
    )
    def prop_kernel(y_hbm, src_hbm, dst_hbm, out_hbm,
                    src_v, dst_v, b0, b1, b2, b3, acc_sh,
                    g0, g1, g2, g3, s0, s1, s2, s3):
        c = lax.axis_index("c")
        s = lax.axis_index("s")
        bufs, gsems, ssems = [b0, b1, b2, b3], [g0, g1, g2, g3], [s0, s1, s2, s3]
        stage_v = b0.at[pl.ds(0, STAGE_ROWS)]
        for ck in range(kc):

            @pl.when(c == ck // kcpc)
            def _run(ck=ck):
                _stage_rows(lambda r0: y_hbm.at[ck, pl.ds(r0, STAGE_ROWS)],
                            lambda r0: acc_sh.at[pl.ds(r0, STAGE_ROWS)],
                            stage_v, s)
                plsc.subcore_barrier()
                for p in range(4):
                    pltpu.sync_copy(src_hbm.at[s, pl.ds(p * NB, NB)], src_v)
                    pltpu.sync_copy(dst_hbm.at[s, pl.ds(p * NB, NB)], dst_v)
                    _edge_ring(y_hbm.at[ck], src_v, dst_v, bufs, gsems, ssems,
                               acc_sh, NB)
                plsc.subcore_barrier()
                _stage_rows(lambda r0: acc_sh.at[pl.ds(r0, STAGE_ROWS)],
                            lambda r0: out_hbm.at[ck, pl.ds(r0, STAGE_ROWS)],
                            stage_v, s)
                plsc.subcore_barrier()

    return prop_kernel


def _sc_propagate_l2():
    @functools.partial(
        pl.kernel,
        out_type=jax.ShapeDtypeStruct((NC, N, 128), jnp.float32),
        mesh=plsc.VectorSubcoreMesh(**_MESH),
        scratch_types=list(_PROP_SCRATCH),
    )
    def prop_kernel(y_hbm, src_hbm, dst_hbm, out_hbm,
                    src_v, dst_v, b0, b1, b2, b3, acc_sh,
                    g0, g1, g2, g3, s0, s1, s2, s3):
        c = lax.axis_index("c")
        s = lax.axis_index("s")
        wid = c * NS + s
        bufs, gsems, ssems = [b0, b1, b2, b3], [g0, g1, g2, g3], [s0, s1, s2, s3]
        stage_v = b0.at[pl.ds(0, STAGE_ROWS)]
        _stage_rows(lambda r0: y_hbm.at[pl.ds(r0, STAGE_ROWS)],
                    lambda r0: acc_sh.at[pl.ds(r0, STAGE_ROWS)],
                    stage_v, s)
        plsc.subcore_barrier()
        for p in range(2):
            pltpu.sync_copy(src_hbm.at[wid, pl.ds(p * NB, NB)], src_v)
            pltpu.sync_copy(dst_hbm.at[wid, pl.ds(p * NB, NB)], dst_v)
            _edge_ring(y_hbm, src_v, dst_v, bufs, gsems, ssems, acc_sh, NB)
        plsc.subcore_barrier()
        _stage_rows(lambda r0: acc_sh.at[pl.ds(r0, STAGE_ROWS)],
                    lambda r0: out_hbm.at[c, pl.ds(r0, STAGE_ROWS)],
                    stage_v, s)

    return prop_kernel


def _dinv(deg_ref):
    deg = deg_ref[:, 0] + deg_ref[:, 1] + 1.0
    return lax.rsqrt(jnp.maximum(deg, 1.0))


BN = 1000


def _mm1_kernel(x_ref, w_ref, deg_ref, y_ref):
    dinv = _dinv(deg_ref)
    y = jnp.dot(x_ref[...], w_ref[...], preferred_element_type=jnp.float32)
    y_ref[0] = y * dinv[:, None]


def _mid_kernel(acc_ref, deg_ref, b1_ref, w2_ref, h_ref, y2_ref):
    dinv = _dinv(deg_ref)
    hcat = jnp.concatenate([acc_ref[k] for k in range(4)], axis=1)
    h = jax.nn.relu(hcat * dinv[:, None] + b1_ref[0, :])
    h_ref[...] = h
    y2 = jnp.dot(h, w2_ref[...], preferred_element_type=jnp.float32)
    y2_ref[...] = y2 * dinv[:, None]


def _fin_kernel(acc_ref, y2_ref, deg_ref, b2_ref, out_ref):
    dinv = _dinv(deg_ref)
    tot = acc_ref[0] + acc_ref[1] - y2_ref[...]
    out_ref[...] = tot * dinv[:, None] + b2_ref[0, :]


def _pad_edges(v, nway, ept, pad_dst):
    npad = ept - E // nway
    per = v.reshape(nway, E // nway)
    if pad_dst:
        pads = N + (jnp.arange(nway * npad, dtype=jnp.int32) % N_PAD_ROWS)
    else:
        pads = jnp.arange(nway * npad, dtype=jnp.int32) % N
    return jnp.concatenate([per, pads.reshape(nway, npad)], axis=1)


def kernel(x, edge_index, W1, b1, W2, b2):
    src = edge_index[0].astype(jnp.int32)
    dst = edge_index[1].astype(jnp.int32)
    src16 = _pad_edges(src, 16, EPT16, False).reshape(16, 4 * NB, EB)
    dst16 = _pad_edges(dst, 16, EPT16, True).reshape(16, 4 * NB, EB)
    src32 = _pad_edges(src, 32, EPT32, False).reshape(32, 2 * NB, EB)
    dst32 = _pad_edges(dst, 32, EPT32, True).reshape(32, 2 * NB, EB)
    dstd = _pad_edges(dst, 32, EPT32, True).reshape(32, EPT32 // DB, DB)

    degp = _sc_degree()(dstd)
    degp = degp.reshape(2, N_ACC).T

    y1 = pl.pallas_call(
        _mm1_kernel,
        grid=(N // BN, HIDDEN // 128),
        in_specs=[
            pl.BlockSpec((BN, IN_DIM), lambda i, j: (i, 0)),
            pl.BlockSpec((IN_DIM, 128), lambda i, j: (0, j)),
            pl.BlockSpec((BN, NC), lambda i, j: (i, 0)),
        ],
        out_specs=pl.BlockSpec((1, BN, 128), lambda i, j: (j, i, 0)),
        out_shape=jax.ShapeDtypeStruct((4, N, 128), jnp.float32),
    )(x, W1, degp)

    acc1 = _sc_propagate_l1()(y1, src16, dst16)

    h, y2 = pl.pallas_call(
        _mid_kernel,
        grid=(N // BN,),
        in_specs=[
            pl.BlockSpec((4, BN, 128), lambda i: (0, i, 0)),
            pl.BlockSpec((BN, NC), lambda i: (i, 0)),
            pl.BlockSpec((1, HIDDEN), lambda i: (0, 0)),
            pl.BlockSpec((HIDDEN, NUM_CLASSES), lambda i: (0, 0)),
        ],
        out_specs=[
            pl.BlockSpec((BN, HIDDEN), lambda i: (i, 0)),
            pl.BlockSpec((BN, NUM_CLASSES), lambda i: (i, 0)),
        ],
        out_shape=[
            jax.ShapeDtypeStruct((N, HIDDEN), jnp.float32),
            jax.ShapeDtypeStruct((N, NUM_CLASSES), jnp.float32),
        ],
    )(acc1, degp, b1.reshape(1, HIDDEN), W2)

    acc2 = _sc_propagate_l2()(y2, src32, dst32)

    logits = pl.pallas_call(
        _fin_kernel,
        grid=(N // BN,),
        in_specs=[
            pl.BlockSpec((NC, BN, 128), lambda i: (0, i, 0)),
            pl.BlockSpec((BN, NUM_CLASSES), lambda i: (i, 0)),
            pl.BlockSpec((BN, NC), lambda i: (i, 0)),
            pl.BlockSpec((1, NUM_CLASSES), lambda i: (0, 0)),
        ],
        out_specs=pl.BlockSpec((BN, NUM_CLASSES), lambda i: (i, 0)),
        out_shape=jax.ShapeDtypeStruct((N, NUM_CLASSES), jnp.float32),
    )(acc2, y2, degp, b2.reshape(1, NUM_CLASSES))

    return (logits, h)


# speedup vs baseline: 1.0615x; 1.0615x over previous
import functools

import jax
import jax.numpy as jnp
from jax import lax
from jax.experimental import pallas as pl
from jax.experimental.pallas import tpu as pltpu
from jax.experimental.pallas import tpu_sc as plsc

N = 10000
E = 160000
IN_DIM = 256
HIDDEN = 512
NUM_CLASSES = 128

NC = 2
NS = 16
BATCH = 128
N_ACC = 10240
N_PAD_ROWS = N_ACC - N
E_PAD = 16 * 80 * BATCH
NB32 = E_PAD // (32 * BATCH)
NB16 = E_PAD // (16 * BATCH)

STRIP = N_ACC // NS
STAGE_ROWS = 80

_MESH = dict(core_axis_name="c", subcore_axis_name="s")


def _row_span(s):
    start = s * STRIP
    nch = jnp.where(s < NS - 1, STRIP // STAGE_ROWS,
                    (N - (NS - 1) * STRIP) // STAGE_ROWS)
    return start, nch


def _sc_degree():
    @functools.partial(
        pl.kernel,
        out_type=jax.ShapeDtypeStruct((NC * N_ACC,), jnp.float32),
        mesh=plsc.VectorSubcoreMesh(**_MESH),
        scratch_types=[
            pltpu.VMEM((NB32, BATCH), jnp.int32),
            pltpu.VMEM((BATCH,), jnp.float32),
            pltpu.VMEM((STRIP,), jnp.float32),
            pltpu.VMEM_SHARED((N_ACC,), jnp.float32),
        ],
    )
    def deg_kernel(dst_hbm, out_hbm, idx_v, ones_v, stage_v, acc_sh):
        c = lax.axis_index("c")
        s = lax.axis_index("s")
        wid = c * NS + s
        for i in range(BATCH // 16):
            ones_v[pl.ds(i * 16, 16)] = jnp.ones((16,), jnp.float32)
        for i in range(STRIP // 16):
            stage_v[pl.ds(i * 16, 16)] = jnp.zeros((16,), jnp.float32)
        pltpu.sync_copy(dst_hbm.at[wid], idx_v)
        pltpu.sync_copy(stage_v, acc_sh.at[pl.ds(s * STRIP, STRIP)])
        plsc.subcore_barrier()

        def body(b, carry):
            pltpu.sync_copy(ones_v, acc_sh.at[idx_v.at[b]], add=True)
            return carry

        lax.fori_loop(0, NB32, body, 0)
        plsc.subcore_barrier()
        pltpu.sync_copy(acc_sh.at[pl.ds(s * STRIP, STRIP)], stage_v)
        pltpu.sync_copy(stage_v, out_hbm.at[pl.ds(c * N_ACC + s * STRIP, STRIP)])

    return deg_kernel


def _edge_pipeline(y_view, src_v, dst_v, gbuf0, gbuf1, gsem0, gsem1,
                   ssem0, ssem1, acc_sh, nb):
    pltpu.async_copy(y_view.at[src_v.at[0]], gbuf0, gsem0)

    def body(j, carry):
        b0 = 2 * j
        b1 = b0 + 1
        pltpu.make_async_copy(y_view.at[src_v.at[b0]], gbuf0, gsem0).wait()

        @pl.when(j > 0)
        def _drain1():
            pltpu.make_async_copy(
                gbuf1, acc_sh.at[dst_v.at[b0 - 1]], ssem1).wait()

        pltpu.async_copy(y_view.at[src_v.at[b1]], gbuf1, gsem1)
        pltpu.async_copy(gbuf0, acc_sh.at[dst_v.at[b0]], ssem0, add=True)
        pltpu.make_async_copy(y_view.at[src_v.at[b1]], gbuf1, gsem1).wait()
        pltpu.make_async_copy(gbuf0, acc_sh.at[dst_v.at[b0]], ssem0).wait()

        @pl.when(b1 + 1 < nb)
        def _next():
            pltpu.async_copy(y_view.at[src_v.at[b1 + 1]], gbuf0, gsem0)

        pltpu.async_copy(gbuf1, acc_sh.at[dst_v.at[b1]], ssem1, add=True)
        return carry

    lax.fori_loop(0, nb // 2, body, 0)
    pltpu.make_async_copy(gbuf1, acc_sh.at[dst_v.at[nb - 1]], ssem1).wait()


def _stage_rows(src_at_rows, dst_at_rows, stage_v, s):
    start, nch = _row_span(s)

    def body(r, carry):
        row0 = start + r * STAGE_ROWS
        pltpu.sync_copy(src_at_rows(row0), stage_v)
        pltpu.sync_copy(stage_v, dst_at_rows(row0))
        return carry

    lax.fori_loop(0, nch, body, 0)


def _sc_propagate_l1():
    kc, cw, kcpc = 4, 128, 2

    @functools.partial(
        pl.kernel,
        out_type=jax.ShapeDtypeStruct((kc, N, cw), jnp.float32),
        mesh=plsc.VectorSubcoreMesh(**_MESH),
        scratch_types=[
            pltpu.VMEM((NB32, BATCH), jnp.int32),
            pltpu.VMEM((NB32, BATCH), jnp.int32),
            pltpu.VMEM((BATCH, cw), jnp.float32),
            pltpu.VMEM((BATCH, cw), jnp.float32),
            pltpu.VMEM_SHARED((N_ACC, cw), jnp.float32),
            pltpu.SemaphoreType.DMA,
            pltpu.SemaphoreType.DMA,
            pltpu.SemaphoreType.DMA,
            pltpu.SemaphoreType.DMA,
        ],
    )
    def prop_kernel(y_hbm, src_hbm, dst_hbm, out_hbm,
                    src_v, dst_v, gbuf0, gbuf1, acc_sh,
                    gsem0, gsem1, ssem0, ssem1):
        c = lax.axis_index("c")
        s = lax.axis_index("s")
        stage_v = gbuf0.at[pl.ds(0, STAGE_ROWS)]
        for ck in range(kc):

            @pl.when(c == ck // kcpc)
            def _run(ck=ck):
                _stage_rows(lambda r0: y_hbm.at[ck, pl.ds(r0, STAGE_ROWS)],
                            lambda r0: acc_sh.at[pl.ds(r0, STAGE_ROWS)],
                            stage_v, s)
                plsc.subcore_barrier()
                for p in range(2):
                    pltpu.sync_copy(src_hbm.at[s, pl.ds(p * NB32, NB32)], src_v)
                    pltpu.sync_copy(dst_hbm.at[s, pl.ds(p * NB32, NB32)], dst_v)
                    _edge_pipeline(y_hbm.at[ck], src_v, dst_v, gbuf0, gbuf1,
                                   gsem0, gsem1, ssem0, ssem1, acc_sh, NB32)
                plsc.subcore_barrier()
                _stage_rows(lambda r0: acc_sh.at[pl.ds(r0, STAGE_ROWS)],
                            lambda r0: out_hbm.at[ck, pl.ds(r0, STAGE_ROWS)],
                            stage_v, s)
                plsc.subcore_barrier()

    return prop_kernel


def _sc_propagate_l2():
    cw = 128

    @functools.partial(
        pl.kernel,
        out_type=jax.ShapeDtypeStruct((NC, N, cw), jnp.float32),
        mesh=plsc.VectorSubcoreMesh(**_MESH),
        scratch_types=[
            pltpu.VMEM((NB32, BATCH), jnp.int32),
            pltpu.VMEM((NB32, BATCH), jnp.int32),
            pltpu.VMEM((BATCH, cw), jnp.float32),
            pltpu.VMEM((BATCH, cw), jnp.float32),
            pltpu.VMEM_SHARED((N_ACC, cw), jnp.float32),
            pltpu.SemaphoreType.DMA,
            pltpu.SemaphoreType.DMA,
            pltpu.SemaphoreType.DMA,
            pltpu.SemaphoreType.DMA,
        ],
    )
    def prop_kernel(y_hbm, src_hbm, dst_hbm, out_hbm,
                    src_v, dst_v, gbuf0, gbuf1, acc_sh,
                    gsem0, gsem1, ssem0, ssem1):
        c = lax.axis_index("c")
        s = lax.axis_index("s")
        wid = c * NS + s
        stage_v = gbuf0.at[pl.ds(0, STAGE_ROWS)]
        pltpu.sync_copy(src_hbm.at[wid], src_v)
        pltpu.sync_copy(dst_hbm.at[wid], dst_v)
        _stage_rows(lambda r0: y_hbm.at[pl.ds(r0, STAGE_ROWS)],
                    lambda r0: acc_sh.at[pl.ds(r0, STAGE_ROWS)],
                    stage_v, s)
        plsc.subcore_barrier()
        _edge_pipeline(y_hbm, src_v, dst_v, gbuf0, gbuf1,
                       gsem0, gsem1, ssem0, ssem1, acc_sh, NB32)
        plsc.subcore_barrier()
        _stage_rows(lambda r0: acc_sh.at[pl.ds(r0, STAGE_ROWS)],
                    lambda r0: out_hbm.at[c, pl.ds(r0, STAGE_ROWS)],
                    stage_v, s)

    return prop_kernel


def _dinv(deg_ref):
    deg = deg_ref[0, :] + deg_ref[1, :] + 1.0
    return lax.rsqrt(jnp.maximum(deg, 1.0))


BN = 1024


def _mm1_kernel(x_ref, w_ref, deg_ref, y_ref):
    dinv = _dinv(deg_ref)
    y = jnp.dot(x_ref[...], w_ref[...], preferred_element_type=jnp.float32)
    y = y * dinv[:, None]
    y_ref[0] = y[:, :128]
    y_ref[1] = y[:, 128:]


def _mid_kernel(acc_ref, deg_ref, b1_ref, w2_ref, h_ref, y2_ref):
    dinv = _dinv(deg_ref)
    hcat = jnp.concatenate([acc_ref[k] for k in range(4)], axis=1)
    h = jax.nn.relu(hcat * dinv[:, None] + b1_ref[0, :])
    h_ref[...] = h
    y2 = jnp.dot(h, w2_ref[...], preferred_element_type=jnp.float32)
    y2_ref[...] = y2 * dinv[:, None]


def _fin_kernel(acc_ref, y2_ref, deg_ref, b2_ref, out_ref):
    dinv = _dinv(deg_ref)
    tot = acc_ref[0] + acc_ref[1] - y2_ref[...]
    out_ref[...] = tot * dinv[:, None] + b2_ref[0, :]


def kernel(x, edge_index, W1, b1, W2, b2):
    src = edge_index[0].astype(jnp.int32)
    dst = edge_index[1].astype(jnp.int32)
    npad = E_PAD - E
    pad_src = jnp.arange(npad, dtype=jnp.int32) % N
    pad_dst = N + jnp.arange(npad, dtype=jnp.int32) % N_PAD_ROWS
    srcp = jnp.concatenate([src, pad_src])
    dstp = jnp.concatenate([dst, pad_dst])
    dst32 = dstp.reshape(32, NB32, BATCH)
    src32 = srcp.reshape(32, NB32, BATCH)
    src16 = srcp.reshape(16, NB16, BATCH)
    dst16 = dstp.reshape(16, NB16, BATCH)

    degp = _sc_degree()(dst32)
    degp = degp.reshape(2, N_ACC)

    y1 = pl.pallas_call(
        _mm1_kernel,
        grid=(10, HIDDEN // 256),
        in_specs=[
            pl.BlockSpec((BN, IN_DIM), lambda i, j: (i, 0)),
            pl.BlockSpec((IN_DIM, 256), lambda i, j: (0, j)),
            pl.BlockSpec((NC, BN), lambda i, j: (0, i)),
        ],
        out_specs=pl.BlockSpec((2, BN, 128), lambda i, j: (j, i, 0)),
        out_shape=jax.ShapeDtypeStruct((4, N, 128), jnp.float32),
    )(x, W1, degp)

    acc1 = _sc_propagate_l1()(y1, src16, dst16)

    h, y2 = pl.pallas_call(
        _mid_kernel,
        grid=(10,),
        in_specs=[
            pl.BlockSpec((4, BN, 128), lambda i: (0, i, 0)),
            pl.BlockSpec((NC, BN), lambda i: (0, i)),
            pl.BlockSpec((1, HIDDEN), lambda i: (0, 0)),
            pl.BlockSpec((HIDDEN, NUM_CLASSES), lambda i: (0, 0)),
        ],
        out_specs=[
            pl.BlockSpec((BN, HIDDEN), lambda i: (i, 0)),
            pl.BlockSpec((BN, NUM_CLASSES), lambda i: (i, 0)),
        ],
        out_shape=[
            jax.ShapeDtypeStruct((N, HIDDEN), jnp.float32),
            jax.ShapeDtypeStruct((N, NUM_CLASSES), jnp.float32),
        ],
    )(acc1, degp, b1.reshape(1, HIDDEN), W2)

    acc2 = _sc_propagate_l2()(y2, src32, dst32)

    logits = pl.pallas_call(
        _fin_kernel,
        grid=(10,),
        in_specs=[
            pl.BlockSpec((NC, BN, 128), lambda i: (0, i, 0)),
            pl.BlockSpec((BN, NUM_CLASSES), lambda i: (i, 0)),
            pl.BlockSpec((NC, BN), lambda i: (0, i)),
            pl.BlockSpec((1, NUM_CLASSES), lambda i: (0, 0)),
        ],
        out_specs=pl.BlockSpec((BN, NUM_CLASSES), lambda i: (i, 0)),
        out_shape=jax.ShapeDtypeStruct((N, NUM_CLASSES), jnp.float32),
    )(acc2, y2, degp, b2.reshape(1, NUM_CLASSES))

    return (logits, h)


# speedup vs baseline: 1.0935x; 1.0302x over previous
import functools

import jax
import jax.numpy as jnp
from jax import lax
from jax.experimental import pallas as pl
from jax.experimental.pallas import tpu as pltpu
from jax.experimental.pallas import tpu_sc as plsc

N = 10000
E = 160000
IN_DIM = 256
HIDDEN = 512
NUM_CLASSES = 128

NC = 2
NS = 16
BATCH = 128
N_ACC = 10240
N_PAD_ROWS = N_ACC - N
E_PAD = 16 * 80 * BATCH
NB32 = E_PAD // (32 * BATCH)
NB16 = E_PAD // (16 * BATCH)

STRIP = N_ACC // NS
STAGE_ROWS = 80

_MESH = dict(core_axis_name="c", subcore_axis_name="s")


def _row_span(s):
    start = s * STRIP
    nch = jnp.where(s < NS - 1, STRIP // STAGE_ROWS,
                    (N - (NS - 1) * STRIP) // STAGE_ROWS)
    return start, nch


def _sc_degree():
    @functools.partial(
        pl.kernel,
        out_type=jax.ShapeDtypeStruct((NC * N_ACC,), jnp.float32),
        mesh=plsc.VectorSubcoreMesh(**_MESH),
        scratch_types=[
            pltpu.VMEM((NB32, BATCH), jnp.int32),
            pltpu.VMEM((BATCH,), jnp.float32),
            pltpu.VMEM((STRIP,), jnp.float32),
            pltpu.VMEM_SHARED((N_ACC,), jnp.float32),
        ],
    )
    def deg_kernel(dst_hbm, out_hbm, idx_v, ones_v, stage_v, acc_sh):
        c = lax.axis_index("c")
        s = lax.axis_index("s")
        wid = c * NS + s
        for i in range(BATCH // 16):
            ones_v[pl.ds(i * 16, 16)] = jnp.ones((16,), jnp.float32)
        for i in range(STRIP // 16):
            stage_v[pl.ds(i * 16, 16)] = jnp.zeros((16,), jnp.float32)
        pltpu.sync_copy(dst_hbm.at[wid], idx_v)
        pltpu.sync_copy(stage_v, acc_sh.at[pl.ds(s * STRIP, STRIP)])
        plsc.subcore_barrier()

        def body(b, carry):
            pltpu.sync_copy(ones_v, acc_sh.at[idx_v.at[b]], add=True)
            return carry

        lax.fori_loop(0, NB32, body, 0)
        plsc.subcore_barrier()
        pltpu.sync_copy(acc_sh.at[pl.ds(s * STRIP, STRIP)], stage_v)
        pltpu.sync_copy(stage_v, out_hbm.at[pl.ds(c * N_ACC + s * STRIP, STRIP)])

    return deg_kernel


def _edge_pipeline(y_view, src_v, dst_v, gbuf0, gbuf1, gsem0, gsem1,
                   ssem0, ssem1, acc_sh, nb):
    pltpu.async_copy(y_view.at[src_v.at[0]], gbuf0, gsem0)

    def body(j, carry):
        b0 = 2 * j
        b1 = b0 + 1
        pltpu.make_async_copy(y_view.at[src_v.at[b0]], gbuf0, gsem0).wait()

        @pl.when(j > 0)
        def _drain1():
            pltpu.make_async_copy(
                gbuf1, acc_sh.at[dst_v.at[b0 - 1]], ssem1).wait()

        pltpu.async_copy(y_view.at[src_v.at[b1]], gbuf1, gsem1)
        pltpu.async_copy(gbuf0, acc_sh.at[dst_v.at[b0]], ssem0, add=True)
        pltpu.make_async_copy(y_view.at[src_v.at[b1]], gbuf1, gsem1).wait()
        pltpu.make_async_copy(gbuf0, acc_sh.at[dst_v.at[b0]], ssem0).wait()

        @pl.when(b1 + 1 < nb)
        def _next():
            pltpu.async_copy(y_view.at[src_v.at[b1 + 1]], gbuf0, gsem0)

        pltpu.async_copy(gbuf1, acc_sh.at[dst_v.at[b1]], ssem1, add=True)
        return carry

    lax.fori_loop(0, nb // 2, body, 0)
    pltpu.make_async_copy(gbuf1, acc_sh.at[dst_v.at[nb - 1]], ssem1).wait()


def _stage_rows(src_at_rows, dst_at_rows, stage_v, s):
    del stage_v
    start, nch = _row_span(s)

    def body(r, carry):
        row0 = start + r * STAGE_ROWS
        pltpu.sync_copy(src_at_rows(row0), dst_at_rows(row0))
        return carry

    lax.fori_loop(0, nch, body, 0)


def _sc_propagate_l1():
    kc, cw, kcpc = 4, 128, 2

    @functools.partial(
        pl.kernel,
        out_type=jax.ShapeDtypeStruct((kc, N, cw), jnp.float32),
        mesh=plsc.VectorSubcoreMesh(**_MESH),
        scratch_types=[
            pltpu.VMEM((NB32, BATCH), jnp.int32),
            pltpu.VMEM((NB32, BATCH), jnp.int32),
            pltpu.VMEM((BATCH, cw), jnp.float32),
            pltpu.VMEM((BATCH, cw), jnp.float32),
            pltpu.VMEM_SHARED((N_ACC, cw), jnp.float32),
            pltpu.SemaphoreType.DMA,
            pltpu.SemaphoreType.DMA,
            pltpu.SemaphoreType.DMA,
            pltpu.SemaphoreType.DMA,
        ],
    )
    def prop_kernel(y_hbm, src_hbm, dst_hbm, out_hbm,
                    src_v, dst_v, gbuf0, gbuf1, acc_sh,
                    gsem0, gsem1, ssem0, ssem1):
        c = lax.axis_index("c")
        s = lax.axis_index("s")
        stage_v = gbuf0.at[pl.ds(0, STAGE_ROWS)]
        for ck in range(kc):

            @pl.when(c == ck // kcpc)
            def _run(ck=ck):
                _stage_rows(lambda r0: y_hbm.at[ck, pl.ds(r0, STAGE_ROWS)],
                            lambda r0: acc_sh.at[pl.ds(r0, STAGE_ROWS)],
                            stage_v, s)
                plsc.subcore_barrier()
                for p in range(2):
                    pltpu.sync_copy(src_hbm.at[s, pl.ds(p * NB32, NB32)], src_v)
                    pltpu.sync_copy(dst_hbm.at[s, pl.ds(p * NB32, NB32)], dst_v)
                    _edge_pipeline(y_hbm.at[ck], src_v, dst_v, gbuf0, gbuf1,
                                   gsem0, gsem1, ssem0, ssem1, acc_sh, NB32)
                plsc.subcore_barrier()
                _stage_rows(lambda r0: acc_sh.at[pl.ds(r0, STAGE_ROWS)],
                            lambda r0: out_hbm.at[ck, pl.ds(r0, STAGE_ROWS)],
                            stage_v, s)
                plsc.subcore_barrier()

    return prop_kernel


def _sc_propagate_l2():
    cw = 128

    @functools.partial(
        pl.kernel,
        out_type=jax.ShapeDtypeStruct((NC, N, cw), jnp.float32),
        mesh=plsc.VectorSubcoreMesh(**_MESH),
        scratch_types=[
            pltpu.VMEM((NB32, BATCH), jnp.int32),
            pltpu.VMEM((NB32, BATCH), jnp.int32),
            pltpu.VMEM((BATCH, cw), jnp.float32),
            pltpu.VMEM((BATCH, cw), jnp.float32),
            pltpu.VMEM_SHARED((N_ACC, cw), jnp.float32),
            pltpu.SemaphoreType.DMA,
            pltpu.SemaphoreType.DMA,
            pltpu.SemaphoreType.DMA,
            pltpu.SemaphoreType.DMA,
        ],
    )
    def prop_kernel(y_hbm, src_hbm, dst_hbm, out_hbm,
                    src_v, dst_v, gbuf0, gbuf1, acc_sh,
                    gsem0, gsem1, ssem0, ssem1):
        c = lax.axis_index("c")
        s = lax.axis_index("s")
        wid = c * NS + s
        stage_v = gbuf0.at[pl.ds(0, STAGE_ROWS)]
        pltpu.sync_copy(src_hbm.at[wid], src_v)
        pltpu.sync_copy(dst_hbm.at[wid], dst_v)
        _stage_rows(lambda r0: y_hbm.at[pl.ds(r0, STAGE_ROWS)],
                    lambda r0: acc_sh.at[pl.ds(r0, STAGE_ROWS)],
                    stage_v, s)
        plsc.subcore_barrier()
        _edge_pipeline(y_hbm, src_v, dst_v, gbuf0, gbuf1,
                       gsem0, gsem1, ssem0, ssem1, acc_sh, NB32)
        plsc.subcore_barrier()
        _stage_rows(lambda r0: acc_sh.at[pl.ds(r0, STAGE_ROWS)],
                    lambda r0: out_hbm.at[c, pl.ds(r0, STAGE_ROWS)],
                    stage_v, s)

    return prop_kernel


def _dinv(deg_ref):
    deg = deg_ref[0, :] + deg_ref[1, :] + 1.0
    return lax.rsqrt(jnp.maximum(deg, 1.0))


BN = 1024


def _mm1_kernel(x_ref, w_ref, deg_ref, y_ref):
    dinv = _dinv(deg_ref)
    y = jnp.dot(x_ref[...], w_ref[...], preferred_element_type=jnp.float32)
    y = y * dinv[:, None]
    y_ref[0] = y[:, :128]
    y_ref[1] = y[:, 128:]


def _mid_kernel(acc_ref, deg_ref, b1_ref, w2_ref, h_ref, y2_ref):
    dinv = _dinv(deg_ref)
    hcat = jnp.concatenate([acc_ref[k] for k in range(4)], axis=1)
    h = jax.nn.relu(hcat * dinv[:, None] + b1_ref[0, :])
    h_ref[...] = h
    y2 = jnp.dot(h, w2_ref[...], preferred_element_type=jnp.float32)
    y2_ref[...] = y2 * dinv[:, None]


def _fin_kernel(acc_ref, y2_ref, deg_ref, b2_ref, out_ref):
    dinv = _dinv(deg_ref)
    tot = acc_ref[0] + acc_ref[1] - y2_ref[...]
    out_ref[...] = tot * dinv[:, None] + b2_ref[0, :]


def kernel(x, edge_index, W1, b1, W2, b2):
    src = edge_index[0].astype(jnp.int32)
    dst = edge_index[1].astype(jnp.int32)
    npad = E_PAD - E
    pad_src = jnp.arange(npad, dtype=jnp.int32) % N
    pad_dst = N + jnp.arange(npad, dtype=jnp.int32) % N_PAD_ROWS
    srcp = jnp.concatenate([src, pad_src])
    dstp = jnp.concatenate([dst, pad_dst])
    dst32 = dstp.reshape(32, NB32, BATCH)
    src32 = srcp.reshape(32, NB32, BATCH)
    src16 = srcp.reshape(16, NB16, BATCH)
    dst16 = dstp.reshape(16, NB16, BATCH)

    degp = _sc_degree()(dst32)
    degp = degp.reshape(2, N_ACC)

    y1 = pl.pallas_call(
        _mm1_kernel,
        grid=(10, HIDDEN // 256),
        in_specs=[
            pl.BlockSpec((BN, IN_DIM), lambda i, j: (i, 0)),
            pl.BlockSpec((IN_DIM, 256), lambda i, j: (0, j)),
            pl.BlockSpec((NC, BN), lambda i, j: (0, i)),
        ],
        out_specs=pl.BlockSpec((2, BN, 128), lambda i, j: (j, i, 0)),
        out_shape=jax.ShapeDtypeStruct((4, N, 128), jnp.float32),
    )(x, W1, degp)

    acc1 = _sc_propagate_l1()(y1, src16, dst16)

    h, y2 = pl.pallas_call(
        _mid_kernel,
        grid=(10,),
        in_specs=[
            pl.BlockSpec((4, BN, 128), lambda i: (0, i, 0)),
            pl.BlockSpec((NC, BN), lambda i: (0, i)),
            pl.BlockSpec((1, HIDDEN), lambda i: (0, 0)),
            pl.BlockSpec((HIDDEN, NUM_CLASSES), lambda i: (0, 0)),
        ],
        out_specs=[
            pl.BlockSpec((BN, HIDDEN), lambda i: (i, 0)),
            pl.BlockSpec((BN, NUM_CLASSES), lambda i: (i, 0)),
        ],
        out_shape=[
            jax.ShapeDtypeStruct((N, HIDDEN), jnp.float32),
            jax.ShapeDtypeStruct((N, NUM_CLASSES), jnp.float32),
        ],
    )(acc1, degp, b1.reshape(1, HIDDEN), W2)

    acc2 = _sc_propagate_l2()(y2, src32, dst32)

    logits = pl.pallas_call(
        _fin_kernel,
        grid=(10,),
        in_specs=[
            pl.BlockSpec((NC, BN, 128), lambda i: (0, i, 0)),
            pl.BlockSpec((BN, NUM_CLASSES), lambda i: (i, 0)),
            pl.BlockSpec((NC, BN), lambda i: (0, i)),
            pl.BlockSpec((1, NUM_CLASSES), lambda i: (0, 0)),
        ],
        out_specs=pl.BlockSpec((BN, NUM_CLASSES), lambda i: (i, 0)),
        out_shape=jax.ShapeDtypeStruct((N, NUM_CLASSES), jnp.float32),
    )(acc2, y2, degp, b2.reshape(1, NUM_CLASSES))

    return (logits, h)


# speedup vs baseline: 1.1252x; 1.0290x over previous
import functools

import jax
import jax.numpy as jnp
from jax import lax
from jax.experimental import pallas as pl
from jax.experimental.pallas import tpu as pltpu
from jax.experimental.pallas import tpu_sc as plsc

N = 10000
E = 160000
IN_DIM = 256
HIDDEN = 512
NUM_CLASSES = 128

NC = 2
NS = 16
BATCH = 128
N_ACC = 10240
N_PAD_ROWS = N_ACC - N
E_PAD = 16 * 80 * BATCH
NB32 = E_PAD // (32 * BATCH)
NB16 = E_PAD // (16 * BATCH)

STRIP = N_ACC // NS
STAGE_ROWS = 80

_MESH = dict(core_axis_name="c", subcore_axis_name="s")


def _row_span(s):
    start = s * STRIP
    nch = jnp.where(s < NS - 1, STRIP // STAGE_ROWS,
                    (N - (NS - 1) * STRIP) // STAGE_ROWS)
    return start, nch


def _sc_degree():
    @functools.partial(
        pl.kernel,
        out_type=jax.ShapeDtypeStruct((NC * N_ACC,), jnp.float32),
        mesh=plsc.VectorSubcoreMesh(**_MESH),
        scratch_types=[
            pltpu.VMEM((NB32, BATCH), jnp.int32),
            pltpu.VMEM((BATCH,), jnp.float32),
            pltpu.VMEM((STRIP,), jnp.float32),
            pltpu.VMEM_SHARED((N_ACC,), jnp.float32),
        ],
    )
    def deg_kernel(dst_hbm, out_hbm, idx_v, ones_v, stage_v, acc_sh):
        c = lax.axis_index("c")
        s = lax.axis_index("s")
        wid = c * NS + s
        for i in range(BATCH // 16):
            ones_v[pl.ds(i * 16, 16)] = jnp.ones((16,), jnp.float32)
        for i in range(STRIP // 16):
            stage_v[pl.ds(i * 16, 16)] = jnp.zeros((16,), jnp.float32)
        pltpu.sync_copy(dst_hbm.at[wid], idx_v)
        pltpu.sync_copy(stage_v, acc_sh.at[pl.ds(s * STRIP, STRIP)])
        plsc.subcore_barrier()

        def body(b, carry):
            pltpu.sync_copy(ones_v, acc_sh.at[idx_v.at[b]], add=True)
            return carry

        lax.fori_loop(0, NB32, body, 0)
        plsc.subcore_barrier()
        pltpu.sync_copy(acc_sh.at[pl.ds(s * STRIP, STRIP)], stage_v)
        pltpu.sync_copy(stage_v, out_hbm.at[pl.ds(c * N_ACC + s * STRIP, STRIP)])

    return deg_kernel


def _edge_pipeline(y_view, src_v, dst_v, gbuf0, gbuf1, gsem0, gsem1,
                   ssem0, ssem1, acc_sh, nb):
    pltpu.async_copy(y_view.at[src_v.at[0]], gbuf0, gsem0)

    def body(j, carry):
        b0 = 2 * j
        b1 = b0 + 1
        pltpu.make_async_copy(y_view.at[src_v.at[b0]], gbuf0, gsem0).wait()

        @pl.when(j > 0)
        def _drain1():
            pltpu.make_async_copy(
                gbuf1, acc_sh.at[dst_v.at[b0 - 1]], ssem1).wait()

        pltpu.async_copy(y_view.at[src_v.at[b1]], gbuf1, gsem1)
        pltpu.async_copy(gbuf0, acc_sh.at[dst_v.at[b0]], ssem0, add=True)
        pltpu.make_async_copy(y_view.at[src_v.at[b1]], gbuf1, gsem1).wait()
        pltpu.make_async_copy(gbuf0, acc_sh.at[dst_v.at[b0]], ssem0).wait()

        @pl.when(b1 + 1 < nb)
        def _next():
            pltpu.async_copy(y_view.at[src_v.at[b1 + 1]], gbuf0, gsem0)

        pltpu.async_copy(gbuf1, acc_sh.at[dst_v.at[b1]], ssem1, add=True)
        return carry

    lax.fori_loop(0, nb // 2, body, 0)
    pltpu.make_async_copy(gbuf1, acc_sh.at[dst_v.at[nb - 1]], ssem1).wait()


def _stage_rows(src_at_rows, dst_at_rows, stage_v, s):
    del stage_v
    start, nch = _row_span(s)

    def body(r, carry):
        row0 = start + r * STAGE_ROWS
        pltpu.sync_copy(src_at_rows(row0), dst_at_rows(row0))
        return carry

    lax.fori_loop(0, nch, body, 0)


def _sc_propagate_l1():
    kc, cw, kcpc = 4, 128, 2

    @functools.partial(
        pl.kernel,
        out_type=jax.ShapeDtypeStruct((kc, N, cw), jnp.float32),
        mesh=plsc.VectorSubcoreMesh(**_MESH),
        scratch_types=[
            pltpu.VMEM((NB32, BATCH), jnp.int32),
            pltpu.VMEM((NB32, BATCH), jnp.int32),
            pltpu.VMEM((BATCH, cw), jnp.float32),
            pltpu.VMEM((BATCH, cw), jnp.float32),
            pltpu.VMEM_SHARED((N_ACC, cw), jnp.float32),
            pltpu.SemaphoreType.DMA,
            pltpu.SemaphoreType.DMA,
            pltpu.SemaphoreType.DMA,
            pltpu.SemaphoreType.DMA,
        ],
    )
    def prop_kernel(y_hbm, src_hbm, dst_hbm, out_hbm,
                    src_v, dst_v, gbuf0, gbuf1, acc_sh,
                    gsem0, gsem1, ssem0, ssem1):
        c = lax.axis_index("c")
        s = lax.axis_index("s")
        stage_v = gbuf0.at[pl.ds(0, STAGE_ROWS)]
        for ck in range(kc):

            @pl.when(c == ck // kcpc)
            def _run(ck=ck):
                _stage_rows(lambda r0: y_hbm.at[ck, pl.ds(r0, STAGE_ROWS)],
                            lambda r0: acc_sh.at[pl.ds(r0, STAGE_ROWS)],
                            stage_v, s)
                plsc.subcore_barrier()
                for p in range(2):
                    pltpu.sync_copy(src_hbm.at[s, pl.ds(p * NB32, NB32)], src_v)
                    pltpu.sync_copy(dst_hbm.at[s, pl.ds(p * NB32, NB32)], dst_v)
                    _edge_pipeline(y_hbm.at[ck], src_v, dst_v, gbuf0, gbuf1,
                                   gsem0, gsem1, ssem0, ssem1, acc_sh, NB32)
                plsc.subcore_barrier()
                _stage_rows(lambda r0: acc_sh.at[pl.ds(r0, STAGE_ROWS)],
                            lambda r0: out_hbm.at[ck, pl.ds(r0, STAGE_ROWS)],
                            stage_v, s)
                plsc.subcore_barrier()

    return prop_kernel


def _sc_propagate_l2():
    cw = 128

    @functools.partial(
        pl.kernel,
        out_type=jax.ShapeDtypeStruct((NC, N, cw), jnp.float32),
        mesh=plsc.VectorSubcoreMesh(**_MESH),
        scratch_types=[
            pltpu.VMEM((NB32, BATCH), jnp.int32),
            pltpu.VMEM((NB32, BATCH), jnp.int32),
            pltpu.VMEM((BATCH, cw), jnp.float32),
            pltpu.VMEM((BATCH, cw), jnp.float32),
            pltpu.VMEM_SHARED((N_ACC, cw), jnp.float32),
            pltpu.SemaphoreType.DMA,
            pltpu.SemaphoreType.DMA,
            pltpu.SemaphoreType.DMA,
            pltpu.SemaphoreType.DMA,
        ],
    )
    def prop_kernel(y_hbm, src_hbm, dst_hbm, out_hbm,
                    src_v, dst_v, gbuf0, gbuf1, acc_sh,
                    gsem0, gsem1, ssem0, ssem1):
        c = lax.axis_index("c")
        s = lax.axis_index("s")
        wid = c * NS + s
        stage_v = gbuf0.at[pl.ds(0, STAGE_ROWS)]
        pltpu.sync_copy(src_hbm.at[wid], src_v)
        pltpu.sync_copy(dst_hbm.at[wid], dst_v)
        _stage_rows(lambda r0: y_hbm.at[pl.ds(r0, STAGE_ROWS)],
                    lambda r0: acc_sh.at[pl.ds(r0, STAGE_ROWS)],
                    stage_v, s)
        plsc.subcore_barrier()
        _edge_pipeline(y_hbm, src_v, dst_v, gbuf0, gbuf1,
                       gsem0, gsem1, ssem0, ssem1, acc_sh, NB32)
        plsc.subcore_barrier()
        _stage_rows(lambda r0: acc_sh.at[pl.ds(r0, STAGE_ROWS)],
                    lambda r0: out_hbm.at[c, pl.ds(r0, STAGE_ROWS)],
                    stage_v, s)

    return prop_kernel


def _dinv(deg_ref):
    deg = deg_ref[0, :] + deg_ref[1, :] + 1.0
    return lax.rsqrt(jnp.maximum(deg, 1.0))


BN = 2048


def _mm1_kernel(x_ref, w_ref, deg_ref, y_ref):
    dinv = _dinv(deg_ref)
    y = jnp.dot(x_ref[...], w_ref[...], preferred_element_type=jnp.float32)
    y = y * dinv[:, None]
    y_ref[0] = y[:, :128]
    y_ref[1] = y[:, 128:]


def _mid_kernel(acc_ref, deg_ref, b1_ref, w2_ref, h_ref, y2_ref):
    dinv = _dinv(deg_ref)
    hcat = jnp.concatenate([acc_ref[k] for k in range(4)], axis=1)
    h = jax.nn.relu(hcat * dinv[:, None] + b1_ref[0, :])
    h_ref[...] = h
    y2 = jnp.dot(h, w2_ref[...], preferred_element_type=jnp.float32)
    y2_ref[...] = y2 * dinv[:, None]


def _fin_kernel(acc_ref, y2_ref, deg_ref, b2_ref, out_ref):
    dinv = _dinv(deg_ref)
    tot = acc_ref[0] + acc_ref[1] - y2_ref[...]
    out_ref[...] = tot * dinv[:, None] + b2_ref[0, :]


def kernel(x, edge_index, W1, b1, W2, b2):
    src = edge_index[0].astype(jnp.int32)
    dst = edge_index[1].astype(jnp.int32)
    npad = E_PAD - E
    pad_src = jnp.arange(npad, dtype=jnp.int32) % N
    pad_dst = N + jnp.arange(npad, dtype=jnp.int32) % N_PAD_ROWS
    srcp = jnp.concatenate([src, pad_src])
    dstp = jnp.concatenate([dst, pad_dst])
    dst32 = dstp.reshape(32, NB32, BATCH)
    src32 = srcp.reshape(32, NB32, BATCH)
    src16 = srcp.reshape(16, NB16, BATCH)
    dst16 = dstp.reshape(16, NB16, BATCH)

    degp = _sc_degree()(dst32)
    degp = degp.reshape(2, N_ACC)

    y1 = pl.pallas_call(
        _mm1_kernel,
        grid=(5, HIDDEN // 256),
        in_specs=[
            pl.BlockSpec((BN, IN_DIM), lambda i, j: (i, 0)),
            pl.BlockSpec((IN_DIM, 256), lambda i, j: (0, j)),
            pl.BlockSpec((NC, BN), lambda i, j: (0, i)),
        ],
        out_specs=pl.BlockSpec((2, BN, 128), lambda i, j: (j, i, 0)),
        out_shape=jax.ShapeDtypeStruct((4, N, 128), jnp.float32),
    )(x, W1, degp)

    acc1 = _sc_propagate_l1()(y1, src16, dst16)

    h, y2 = pl.pallas_call(
        _mid_kernel,
        grid=(5,),
        in_specs=[
            pl.BlockSpec((4, BN, 128), lambda i: (0, i, 0)),
            pl.BlockSpec((NC, BN), lambda i: (0, i)),
            pl.BlockSpec((1, HIDDEN), lambda i: (0, 0)),
            pl.BlockSpec((HIDDEN, NUM_CLASSES), lambda i: (0, 0)),
        ],
        out_specs=[
            pl.BlockSpec((BN, HIDDEN), lambda i: (i, 0)),
            pl.BlockSpec((BN, NUM_CLASSES), lambda i: (i, 0)),
        ],
        out_shape=[
            jax.ShapeDtypeStruct((N, HIDDEN), jnp.float32),
            jax.ShapeDtypeStruct((N, NUM_CLASSES), jnp.float32),
        ],
    )(acc1, degp, b1.reshape(1, HIDDEN), W2)

    acc2 = _sc_propagate_l2()(y2, src32, dst32)

    logits = pl.pallas_call(
        _fin_kernel,
        grid=(5,),
        in_specs=[
            pl.BlockSpec((NC, BN, 128), lambda i: (0, i, 0)),
            pl.BlockSpec((BN, NUM_CLASSES), lambda i: (i, 0)),
            pl.BlockSpec((NC, BN), lambda i: (0, i)),
            pl.BlockSpec((1, NUM_CLASSES), lambda i: (0, 0)),
        ],
        out_specs=pl.BlockSpec((BN, NUM_CLASSES), lambda i: (i, 0)),
        out_shape=jax.ShapeDtypeStruct((N, NUM_CLASSES), jnp.float32),
    )(acc2, y2, degp, b2.reshape(1, NUM_CLASSES))

    return (logits, h)
